# baseline (speedup 1.0000x reference)
import jax
import jax.numpy as jnp
from jax import lax
from jax.experimental import pallas as pl
from jax.experimental.pallas import tpu as pltpu
from jax.experimental.pallas import tpu_sc as plsc

EMB = 16
MARGIN = 0.1
NC = 2
NS = 16
NW = NC * NS
TERMS = 32768
TPW = TERMS // NW
CHUNK = 128
NCH = TPW // CHUNK
BLOCKS = TPW // 16


def _sc_body(idx_hbm, ent_hbm, rel_hbm, out_hbm,
             idx_v, hp, rp, tp, hn, rn, tn, sqp_v, sqn_v, sem):
    wid = lax.axis_index("s") * NC + lax.axis_index("c")
    pltpu.sync_copy(idx_hbm.at[wid], idx_v)

    bufs = (hp, rp, tp, hn, rn, tn)
    tables = (ent_hbm, rel_hbm, ent_hbm, ent_hbm, rel_hbm, ent_hbm)
    copies = []
    for j in range(6):
        for c in range(NCH):
            copies.append(pltpu.async_copy(
                tables[j].at[idx_v.at[j, c]],
                bufs[j].at[pl.ds(c * CHUNK, CHUNK)], sem))
    for cp in copies:
        cp.wait()

    iot = lax.iota(jnp.int32, 16)
    perms = [iot ^ s for s in (8, 4, 2, 1)]
    masks = [(iot & s) == 0 for s in (8, 4, 2, 1)]

    dnums = lax.GatherDimensionNumbers(
        offset_dims=(), collapsed_slice_dims=(0,), start_index_map=(0,))

    def _perm(v, pidx):
        return lax.gather(v, pidx[:, None], dnums, (1,),
                          mode=lax.GatherScatterMode.PROMISE_IN_BOUNDS)

    def _rowsums(h, r, t, base):
        es = []
        for j in range(16):
            d = h[base + j, :] + r[base + j, :] - t[base + j, :]
            es.append(d * d)
        for pidx, msk in zip(perms, masks):
            half = len(es) // 2
            es = [jnp.where(msk,
                            es[i] + _perm(es[i], pidx),
                            es[i + half] + _perm(es[i + half], pidx))
                  for i in range(half)]
        return es[0]

    def block(b, carry):
        base = b * 16
        sqp_v[pl.ds(base, 16)] = _rowsums(hp, rp, tp, base)
        sqn_v[pl.ds(base, 16)] = _rowsums(hn, rn, tn, base)
        return carry

    lax.fori_loop(0, BLOCKS, block, jnp.int32(0))
    pltpu.sync_copy(sqp_v, out_hbm.at[0, pl.ds(wid * TPW, TPW)])
    pltpu.sync_copy(sqn_v, out_hbm.at[1, pl.ds(wid * TPW, TPW)])


TBLK = 8192


def _detile_body(xe_ref, xr_ref, oe_ref, or_ref):
    eye = jnp.eye(EMB, dtype=jnp.float32)
    dn = (((0,), (0,)), ((), ()))
    oe_ref[...] = lax.dot_general(xe_ref[...], eye, dn,
                                  preferred_element_type=jnp.float32)
    or_ref[...] = lax.dot_general(xr_ref[...], eye, dn,
                                  preferred_element_type=jnp.float32)


def _loss_body(x_ref, o_ref):
    sp = x_ref[0, :]
    sn = x_ref[1, :]
    loss = jnp.maximum(MARGIN + jnp.sqrt(sp) - jnp.sqrt(sn), 0.0)
    o_ref[...] = jnp.sum(loss).reshape(1, 1)


def kernel(lhs_pos, rhs_pos, lhs_neg, rhs_neg, ent_emb, rel_emb):
    pos = jnp.concatenate([lhs_pos, rhs_pos], axis=0).astype(jnp.int32)
    neg = jnp.concatenate([lhs_neg, rhs_neg], axis=0).astype(jnp.int32)
    allidx = jnp.stack([pos[:, 0], pos[:, 1], pos[:, 2],
                        neg[:, 0], neg[:, 1], neg[:, 2]])
    idx = (allidx.reshape(6, NW, TPW).transpose(1, 0, 2)
           .reshape(NW, 6, NCH, CHUNK))

    num_ent = ent_emb.shape[0]
    grid = pl.cdiv(num_ent, TBLK)
    lin_e, lin_r = pl.pallas_call(
        _detile_body,
        grid=(grid,),
        in_specs=[pl.BlockSpec((EMB, TBLK), lambda i: (0, i)),
                  pl.BlockSpec((EMB, TBLK), lambda i: (0, i))],
        out_specs=[pl.BlockSpec((TBLK, EMB), lambda i: (i, 0)),
                   pl.BlockSpec((TBLK, EMB), lambda i: (i, 0))],
        out_shape=[jax.ShapeDtypeStruct((num_ent, EMB), jnp.float32),
                   jax.ShapeDtypeStruct((num_ent, EMB), jnp.float32)],
    )(ent_emb.T, rel_emb.T)

    mesh = plsc.VectorSubcoreMesh(core_axis_name="c", subcore_axis_name="s")
    sc = pl.kernel(
        _sc_body,
        out_type=jax.ShapeDtypeStruct((2, TERMS), jnp.float32),
        mesh=mesh,
        scratch_types=[
            pltpu.VMEM((6, NCH, CHUNK), jnp.int32),
            pltpu.VMEM((TPW, EMB), jnp.float32),
            pltpu.VMEM((TPW, EMB), jnp.float32),
            pltpu.VMEM((TPW, EMB), jnp.float32),
            pltpu.VMEM((TPW, EMB), jnp.float32),
            pltpu.VMEM((TPW, EMB), jnp.float32),
            pltpu.VMEM((TPW, EMB), jnp.float32),
            pltpu.VMEM((TPW,), jnp.float32),
            pltpu.VMEM((TPW,), jnp.float32),
            pltpu.SemaphoreType.DMA,
        ],
        compiler_params=pltpu.CompilerParams(use_tc_tiling_on_sc=False,
                                             needs_layout_passes=False),
    )
    sq = sc(idx, lin_e, lin_r)

    loss = pl.pallas_call(
        _loss_body,
        out_shape=jax.ShapeDtypeStruct((1, 1), jnp.float32),
    )(sq)
    return loss[0, 0]


# speedup vs baseline: 1.0835x; 1.0835x over previous
import jax
import jax.numpy as jnp
from jax import lax
from jax.experimental import pallas as pl
from jax.experimental.pallas import tpu as pltpu
from jax.experimental.pallas import tpu_sc as plsc

EMB = 16
MARGIN = 0.1
NC = 2
NS = 16
NW = NC * NS
TERMS = 32768
TPW = TERMS // NW
CHUNK = 128
NCH = TPW // CHUNK
BLOCKS = TPW // 16


def _sc_body(idx_hbm, ent_hbm, rel_hbm, out_hbm,
             idx_v, hp, rp, tp, hn, rn, tn, sqp_v, sqn_v, sem):
    wid = lax.axis_index("s") * NC + lax.axis_index("c")
    pltpu.sync_copy(idx_hbm.at[wid], idx_v)

    bufs = (hp, rp, tp, hn, rn, tn)
    tables = (ent_hbm, rel_hbm, ent_hbm, ent_hbm, rel_hbm, ent_hbm)
    copies = []
    for j in range(6):
        for c in range(NCH):
            copies.append(pltpu.async_copy(
                tables[j].at[idx_v.at[j, c]],
                bufs[j].at[pl.ds(c * CHUNK, CHUNK)], sem))
    for cp in copies:
        cp.wait()

    iot = lax.iota(jnp.int32, 16)
    perms = [iot ^ s for s in (8, 4, 2, 1)]
    masks = [(iot & s) == 0 for s in (8, 4, 2, 1)]

    dnums = lax.GatherDimensionNumbers(
        offset_dims=(), collapsed_slice_dims=(0,), start_index_map=(0,))

    def _perm(v, pidx):
        return lax.gather(v, pidx[:, None], dnums, (1,),
                          mode=lax.GatherScatterMode.PROMISE_IN_BOUNDS)

    def _rowsums(h, r, t, base):
        es = []
        for j in range(16):
            d = h[base + j, :] + r[base + j, :] - t[base + j, :]
            es.append(d * d)
        for pidx, msk in zip(perms, masks):
            half = len(es) // 2
            es = [jnp.where(msk,
                            es[i] + _perm(es[i], pidx),
                            es[i + half] + _perm(es[i + half], pidx))
                  for i in range(half)]
        return es[0]

    def block(b, carry):
        base = b * 16
        sqp_v[pl.ds(base, 16)] = _rowsums(hp, rp, tp, base)
        sqn_v[pl.ds(base, 16)] = _rowsums(hn, rn, tn, base)
        return carry

    lax.fori_loop(0, BLOCKS, block, jnp.int32(0))
    pltpu.sync_copy(sqp_v, out_hbm.at[0, pl.ds(wid * TPW, TPW)])
    pltpu.sync_copy(sqn_v, out_hbm.at[1, pl.ds(wid * TPW, TPW)])


TILE_COLS = 7813
PAD_ENT = TILE_COLS * 128
RELAY_STEPS = 245


def _relayout_body(te_hbm, tr_hbm, oe_hbm, or_hbm, tl0, tl1, ob, sem):
    wid = lax.axis_index("s") * NC + lax.axis_index("c")
    i16 = lax.iota(jnp.int32, 16) * 16

    def do(tab, out, t):
        pltpu.sync_copy(tab.at[pl.ds(0, 8), pl.ds(t * 128, 128)], tl0)
        pltpu.sync_copy(tab.at[pl.ds(8, 8), pl.ds(t * 128, 128)], tl1)
        for j in range(8):
            for k in range(8):
                plsc.store_scatter(ob, [i16 + (j * 256 + k)],
                                   tl0[k, pl.ds(j * 16, 16)])
                plsc.store_scatter(ob, [i16 + (j * 256 + k + 8)],
                                   tl1[k, pl.ds(j * 16, 16)])
        pltpu.sync_copy(ob, out.at[pl.ds(t * 2048, 2048)])

    def step(i, carry):
        t = wid + i * NW

        @pl.when(t < TILE_COLS)
        def _():
            do(te_hbm, oe_hbm, t)
            do(tr_hbm, or_hbm, t)

        return carry

    lax.fori_loop(0, RELAY_STEPS, step, jnp.int32(0))


def _loss_body(x_ref, o_ref):
    sp = x_ref[0, :]
    sn = x_ref[1, :]
    loss = jnp.maximum(MARGIN + jnp.sqrt(sp) - jnp.sqrt(sn), 0.0)
    o_ref[...] = jnp.sum(loss).reshape(1, 1)


def kernel(lhs_pos, rhs_pos, lhs_neg, rhs_neg, ent_emb, rel_emb):
    pos = jnp.concatenate([lhs_pos, rhs_pos], axis=0).astype(jnp.int32)
    neg = jnp.concatenate([lhs_neg, rhs_neg], axis=0).astype(jnp.int32)
    allidx = jnp.stack([pos[:, 0], pos[:, 1], pos[:, 2],
                        neg[:, 0], neg[:, 1], neg[:, 2]])
    idx = (allidx.reshape(6, NW, TPW).transpose(1, 0, 2)
           .reshape(NW, 6, NCH, CHUNK))

    mesh = plsc.VectorSubcoreMesh(core_axis_name="c", subcore_axis_name="s")
    relay = pl.kernel(
        _relayout_body,
        out_type=[jax.ShapeDtypeStruct((PAD_ENT * EMB,), jnp.float32),
                  jax.ShapeDtypeStruct((PAD_ENT * EMB,), jnp.float32)],
        mesh=mesh,
        scratch_types=[
            pltpu.VMEM((8, 128), jnp.float32),
            pltpu.VMEM((8, 128), jnp.float32),
            pltpu.VMEM((2048,), jnp.float32),
            pltpu.SemaphoreType.DMA,
        ],
        compiler_params=pltpu.CompilerParams(use_tc_tiling_on_sc=True,
                                             needs_layout_passes=False),
    )
    flat_e, flat_r = relay(ent_emb.T, rel_emb.T)
    lin_e = flat_e.reshape(PAD_ENT, EMB)
    lin_r = flat_r.reshape(PAD_ENT, EMB)
    sc = pl.kernel(
        _sc_body,
        out_type=jax.ShapeDtypeStruct((2, TERMS), jnp.float32),
        mesh=mesh,
        scratch_types=[
            pltpu.VMEM((6, NCH, CHUNK), jnp.int32),
            pltpu.VMEM((TPW, EMB), jnp.float32),
            pltpu.VMEM((TPW, EMB), jnp.float32),
            pltpu.VMEM((TPW, EMB), jnp.float32),
            pltpu.VMEM((TPW, EMB), jnp.float32),
            pltpu.VMEM((TPW, EMB), jnp.float32),
            pltpu.VMEM((TPW, EMB), jnp.float32),
            pltpu.VMEM((TPW,), jnp.float32),
            pltpu.VMEM((TPW,), jnp.float32),
            pltpu.SemaphoreType.DMA,
        ],
        compiler_params=pltpu.CompilerParams(use_tc_tiling_on_sc=False,
                                             needs_layout_passes=False),
    )
    sq = sc(idx, lin_e, lin_r)

    loss = pl.pallas_call(
        _loss_body,
        out_shape=jax.ShapeDtypeStruct((1, 1), jnp.float32),
    )(sq)
    return loss[0, 0]


# speedup vs baseline: 2.9053x; 2.6815x over previous
import jax
import jax.numpy as jnp
from jax import lax
from jax.experimental import pallas as pl
from jax.experimental.pallas import tpu as pltpu
from jax.experimental.pallas import tpu_sc as plsc

EMB = 16
MARGIN = 0.1
NC = 2
NS = 16
NW = NC * NS
TERMS = 32768
TPW = TERMS // NW
CHUNK = 128
NCH = TPW // CHUNK
BLOCKS = TPW // 16


def _sc_body(idx_hbm, ent_hbm, rel_hbm, out_hbm,
             idx_v, hp, rp, tp, hn, rn, tn, sqp_v, sqn_v, sem):
    wid = lax.axis_index("s") * NC + lax.axis_index("c")
    pltpu.sync_copy(idx_hbm.at[wid], idx_v)

    bufs = (hp, rp, tp, hn, rn, tn)
    tables = (ent_hbm, rel_hbm, ent_hbm, ent_hbm, rel_hbm, ent_hbm)
    copies = []
    for j in range(6):
        for c in range(NCH):
            copies.append(pltpu.async_copy(
                tables[j].at[idx_v.at[j, c]],
                bufs[j].at[pl.ds(c * CHUNK, CHUNK)], sem))
    for cp in copies:
        cp.wait()

    iot = lax.iota(jnp.int32, 16)
    perms = [iot ^ s for s in (8, 4, 2, 1)]
    masks = [(iot & s) == 0 for s in (8, 4, 2, 1)]

    dnums = lax.GatherDimensionNumbers(
        offset_dims=(), collapsed_slice_dims=(0,), start_index_map=(0,))

    def _perm(v, pidx):
        return lax.gather(v, pidx[:, None], dnums, (1,),
                          mode=lax.GatherScatterMode.PROMISE_IN_BOUNDS)

    def _rowsums(h, r, t, base):
        es = []
        for j in range(16):
            d = h[base + j, :] + r[base + j, :] - t[base + j, :]
            es.append(d * d)
        for pidx, msk in zip(perms, masks):
            half = len(es) // 2
            es = [jnp.where(msk,
                            es[i] + _perm(es[i], pidx),
                            es[i + half] + _perm(es[i + half], pidx))
                  for i in range(half)]
        return es[0]

    def block(b, carry):
        base = b * 16
        sqp_v[pl.ds(base, 16)] = _rowsums(hp, rp, tp, base)
        sqn_v[pl.ds(base, 16)] = _rowsums(hn, rn, tn, base)
        return carry

    lax.fori_loop(0, BLOCKS, block, jnp.int32(0))
    pltpu.sync_copy(sqp_v, out_hbm.at[0, pl.ds(wid * TPW, TPW)])
    pltpu.sync_copy(sqn_v, out_hbm.at[1, pl.ds(wid * TPW, TPW)])


TILE_COLS = 7813
PAD_ENT = TILE_COLS * 128
GT = 4
GC = GT * 128
FULLG = 1952
TAILC = 576
GPAIRS = 31


def _relayout_body(te_hbm, tr_hbm, oe_hbm, or_hbm,
                   ia0, ia1, ib0, ib1, oba, obb, tin0, tin1, sem, osem):
    wid = lax.axis_index("s") * NC + lax.axis_index("c")
    i16 = lax.iota(jnp.int32, 16) * 16
    ins = ((ia0, ib0), (ia1, ib1))
    obs = (oba, obb)

    def issue_in(tab, g, b):
        c0 = g * GC
        pltpu.async_copy(tab.at[pl.ds(0, 8), pl.ds(c0, GC)], ins[b][0], sem)
        pltpu.async_copy(tab.at[pl.ds(8, 8), pl.ds(c0, GC)], ins[b][1], sem)

    def wait_in(tab, b):
        pltpu.make_async_copy(tab.at[pl.ds(0, 8), pl.ds(0, GC)],
                              ins[b][0], sem).wait()
        pltpu.make_async_copy(tab.at[pl.ds(8, 8), pl.ds(0, GC)],
                              ins[b][1], sem).wait()

    def transpose_tile(src0, src1, dst, tc, jmax):
        for j in range(jmax):
            for k in range(8):
                plsc.store_scatter(dst, [i16 + (tc * 2048 + j * 256 + k)],
                                   src0[k, pl.ds(tc * 128 + j * 16, 16)])
                plsc.store_scatter(dst, [i16 + (tc * 2048 + j * 256 + k + 8)],
                                   src1[k, pl.ds(tc * 128 + j * 16, 16)])

    def do_table(tab, out):
        issue_in(tab, wid, 0)
        issue_in(tab, wid + NW, 1)

        def pair(s2, carry):
            for b in (0, 1):
                g = wid + (2 * s2 + b) * NW

                @pl.when(g < FULLG)
                def _():
                    wait_in(tab, b)

                    @pl.when(2 * s2 + b >= 2)
                    def _():
                        pltpu.make_async_copy(
                            obs[b].at[pl.ds(0, GC * EMB)],
                            out.at[pl.ds(0, GC * EMB)], osem).wait()
                    for tc in range(GT):
                        transpose_tile(ins[b][0], ins[b][1], obs[b], tc, 8)
                    gp = g + 2 * NW

                    @pl.when(gp < FULLG)
                    def _():
                        issue_in(tab, gp, b)
                    pltpu.async_copy(obs[b].at[pl.ds(0, GC * EMB)],
                                     out.at[pl.ds(g * GC * EMB, GC * EMB)],
                                     osem)
            return carry

        lax.fori_loop(0, GPAIRS, pair, jnp.int32(0))
        for b in (0, 1):
            pltpu.make_async_copy(obs[b].at[pl.ds(0, GC * EMB)],
                                  out.at[pl.ds(0, GC * EMB)], osem).wait()

    def do_tail(tab, out):
        c0 = FULLG * GC
        pltpu.sync_copy(tab.at[pl.ds(0, 8), pl.ds(c0, TAILC)], tin0)
        pltpu.sync_copy(tab.at[pl.ds(8, 8), pl.ds(c0, TAILC)], tin1)
        for tc in range(4):
            transpose_tile(tin0, tin1, oba, tc, 8)
        transpose_tile(tin0, tin1, oba, 4, 4)
        pltpu.sync_copy(oba.at[pl.ds(0, TAILC * EMB)],
                        out.at[pl.ds(c0 * EMB, TAILC * EMB)])

    do_table(te_hbm, oe_hbm)
    do_table(tr_hbm, or_hbm)

    @pl.when(wid == 31)
    def _():
        do_tail(te_hbm, oe_hbm)
        do_tail(tr_hbm, or_hbm)


def _loss_body(x_ref, o_ref):
    sp = x_ref[0, :]
    sn = x_ref[1, :]
    loss = jnp.maximum(MARGIN + jnp.sqrt(sp) - jnp.sqrt(sn), 0.0)
    o_ref[...] = jnp.sum(loss).reshape(1, 1)


def kernel(lhs_pos, rhs_pos, lhs_neg, rhs_neg, ent_emb, rel_emb):
    pos = jnp.concatenate([lhs_pos, rhs_pos], axis=0).astype(jnp.int32)
    neg = jnp.concatenate([lhs_neg, rhs_neg], axis=0).astype(jnp.int32)
    allidx = jnp.stack([pos[:, 0], pos[:, 1], pos[:, 2],
                        neg[:, 0], neg[:, 1], neg[:, 2]])
    idx = (allidx.reshape(6, NW, TPW).transpose(1, 0, 2)
           .reshape(NW, 6, NCH, CHUNK))

    mesh = plsc.VectorSubcoreMesh(core_axis_name="c", subcore_axis_name="s")
    relay = pl.kernel(
        _relayout_body,
        out_type=[jax.ShapeDtypeStruct((PAD_ENT * EMB,), jnp.float32),
                  jax.ShapeDtypeStruct((PAD_ENT * EMB,), jnp.float32)],
        mesh=mesh,
        scratch_types=[
            pltpu.VMEM((8, GC), jnp.float32),
            pltpu.VMEM((8, GC), jnp.float32),
            pltpu.VMEM((8, GC), jnp.float32),
            pltpu.VMEM((8, GC), jnp.float32),
            pltpu.VMEM((TAILC * EMB,), jnp.float32),
            pltpu.VMEM((GC * EMB,), jnp.float32),
            pltpu.VMEM((8, TAILC), jnp.float32),
            pltpu.VMEM((8, TAILC), jnp.float32),
            pltpu.SemaphoreType.DMA,
            pltpu.SemaphoreType.DMA,
        ],
        compiler_params=pltpu.CompilerParams(use_tc_tiling_on_sc=True,
                                             needs_layout_passes=False),
    )
    flat_e, flat_r = relay(ent_emb.T, rel_emb.T)
    lin_e = flat_e.reshape(PAD_ENT, EMB)
    lin_r = flat_r.reshape(PAD_ENT, EMB)
    sc = pl.kernel(
        _sc_body,
        out_type=jax.ShapeDtypeStruct((2, TERMS), jnp.float32),
        mesh=mesh,
        scratch_types=[
            pltpu.VMEM((6, NCH, CHUNK), jnp.int32),
            pltpu.VMEM((TPW, EMB), jnp.float32),
            pltpu.VMEM((TPW, EMB), jnp.float32),
            pltpu.VMEM((TPW, EMB), jnp.float32),
            pltpu.VMEM((TPW, EMB), jnp.float32),
            pltpu.VMEM((TPW, EMB), jnp.float32),
            pltpu.VMEM((TPW, EMB), jnp.float32),
            pltpu.VMEM((TPW,), jnp.float32),
            pltpu.VMEM((TPW,), jnp.float32),
            pltpu.SemaphoreType.DMA,
        ],
        compiler_params=pltpu.CompilerParams(use_tc_tiling_on_sc=False,
                                             needs_layout_passes=False),
    )
    sq = sc(idx, lin_e, lin_r)

    loss = pl.pallas_call(
        _loss_body,
        out_shape=jax.ShapeDtypeStruct((1, 1), jnp.float32),
    )(sq)
    return loss[0, 0]


# speedup vs baseline: 2.9260x; 1.0071x over previous
import jax
import jax.numpy as jnp
from jax import lax
from jax.experimental import pallas as pl
from jax.experimental.pallas import tpu as pltpu
from jax.experimental.pallas import tpu_sc as plsc

EMB = 16
MARGIN = 0.1
NC = 2
NS = 16
NW = NC * NS
TERMS = 32768
TPW = TERMS // NW
CHUNK = 128
NCH = TPW // CHUNK
BLOCKS = TPW // 16


def _sc_body(idx_hbm, ent_hbm, rel_hbm, out_hbm,
             idx_v, hp, rp, tp, hn, rn, tn, sqp_v, sqn_v, sem):
    wid = lax.axis_index("s") * NC + lax.axis_index("c")
    pltpu.sync_copy(idx_hbm.at[wid], idx_v)

    bufs = (hp, rp, tp, hn, rn, tn)
    tables = (ent_hbm, rel_hbm, ent_hbm, ent_hbm, rel_hbm, ent_hbm)
    copies = []
    for j in range(6):
        for c in range(NCH):
            copies.append(pltpu.async_copy(
                tables[j].at[idx_v.at[j, c]],
                bufs[j].at[pl.ds(c * CHUNK, CHUNK)], sem))
    for cp in copies:
        cp.wait()

    iot = lax.iota(jnp.int32, 16)
    perms = [iot ^ s for s in (8, 4, 2, 1)]
    masks = [(iot & s) == 0 for s in (8, 4, 2, 1)]

    dnums = lax.GatherDimensionNumbers(
        offset_dims=(), collapsed_slice_dims=(0,), start_index_map=(0,))

    def _perm(v, pidx):
        return lax.gather(v, pidx[:, None], dnums, (1,),
                          mode=lax.GatherScatterMode.PROMISE_IN_BOUNDS)

    def _rowsums(h, r, t, base):
        es = []
        for j in range(16):
            d = h[base + j, :] + r[base + j, :] - t[base + j, :]
            es.append(d * d)
        for pidx, msk in zip(perms, masks):
            half = len(es) // 2
            es = [jnp.where(msk,
                            es[i] + _perm(es[i], pidx),
                            es[i + half] + _perm(es[i + half], pidx))
                  for i in range(half)]
        return es[0]

    def block(b, carry):
        base = b * 16
        sqp_v[pl.ds(base, 16)] = _rowsums(hp, rp, tp, base)
        sqn_v[pl.ds(base, 16)] = _rowsums(hn, rn, tn, base)
        return carry

    lax.fori_loop(0, BLOCKS, block, jnp.int32(0))
    pltpu.sync_copy(sqp_v, out_hbm.at[0, pl.ds(wid * TPW, TPW)])
    pltpu.sync_copy(sqn_v, out_hbm.at[1, pl.ds(wid * TPW, TPW)])


TILE_COLS = 7813
PAD_ENT = TILE_COLS * 128
GT = 4
GC = GT * 128
FULLG = 1952
TAILC = 576
GPAIRS = 31


def _relayout_body(te_hbm, tr_hbm, oe_hbm, or_hbm,
                   ia0, ia1, oba, obb, tin0, sem, osem):
    wid = lax.axis_index("s") * NC + lax.axis_index("c")
    i16 = lax.iota(jnp.int32, 16) * 16
    ins = (ia0, ia1)
    obs = (oba, obb)

    def issue_in(tab, g, b):
        c0 = g * GC
        pltpu.async_copy(tab.at[pl.ds(0, 16), pl.ds(c0, GC)], ins[b], sem)

    def wait_in(tab, b):
        pltpu.make_async_copy(tab.at[pl.ds(0, 16), pl.ds(0, GC)],
                              ins[b], sem).wait()

    def transpose_tile(src, dst, tc, jmax):
        for j in range(jmax):
            for k in range(16):
                plsc.store_scatter(dst, [i16 + (tc * 2048 + j * 256 + k)],
                                   src[k, pl.ds(tc * 128 + j * 16, 16)])

    def do_table(tab, out):
        issue_in(tab, wid, 0)
        issue_in(tab, wid + NW, 1)

        def pair(s2, carry):
            for b in (0, 1):
                g = wid + (2 * s2 + b) * NW

                @pl.when(g < FULLG)
                def _():
                    wait_in(tab, b)

                    @pl.when(2 * s2 + b >= 2)
                    def _():
                        pltpu.make_async_copy(
                            obs[b].at[pl.ds(0, GC * EMB)],
                            out.at[pl.ds(0, GC * EMB)], osem).wait()
                    for tc in range(GT):
                        transpose_tile(ins[b], obs[b], tc, 8)
                    gp = g + 2 * NW

                    @pl.when(gp < FULLG)
                    def _():
                        issue_in(tab, gp, b)
                    pltpu.async_copy(obs[b].at[pl.ds(0, GC * EMB)],
                                     out.at[pl.ds(g * GC * EMB, GC * EMB)],
                                     osem)
            return carry

        lax.fori_loop(0, GPAIRS, pair, jnp.int32(0))
        for b in (0, 1):
            pltpu.make_async_copy(obs[b].at[pl.ds(0, GC * EMB)],
                                  out.at[pl.ds(0, GC * EMB)], osem).wait()

    def do_tail(tab, out):
        c0 = FULLG * GC
        pltpu.sync_copy(tab.at[pl.ds(0, 16), pl.ds(c0, TAILC)], tin0)
        for tc in range(4):
            transpose_tile(tin0, oba, tc, 8)
        transpose_tile(tin0, oba, 4, 4)
        pltpu.sync_copy(oba.at[pl.ds(0, TAILC * EMB)],
                        out.at[pl.ds(c0 * EMB, TAILC * EMB)])

    do_table(te_hbm, oe_hbm)
    do_table(tr_hbm, or_hbm)

    @pl.when(wid == 31)
    def _():
        do_tail(te_hbm, oe_hbm)
        do_tail(tr_hbm, or_hbm)


def _loss_body(x_ref, o_ref):
    sp = x_ref[0, :]
    sn = x_ref[1, :]
    loss = jnp.maximum(MARGIN + jnp.sqrt(sp) - jnp.sqrt(sn), 0.0)
    o_ref[...] = jnp.sum(loss).reshape(1, 1)


def kernel(lhs_pos, rhs_pos, lhs_neg, rhs_neg, ent_emb, rel_emb):
    pos = jnp.concatenate([lhs_pos, rhs_pos], axis=0).astype(jnp.int32)
    neg = jnp.concatenate([lhs_neg, rhs_neg], axis=0).astype(jnp.int32)
    allidx = jnp.stack([pos[:, 0], pos[:, 1], pos[:, 2],
                        neg[:, 0], neg[:, 1], neg[:, 2]])
    idx = (allidx.reshape(6, NW, TPW).transpose(1, 0, 2)
           .reshape(NW, 6, NCH, CHUNK))

    mesh = plsc.VectorSubcoreMesh(core_axis_name="c", subcore_axis_name="s")
    relay = pl.kernel(
        _relayout_body,
        out_type=[jax.ShapeDtypeStruct((PAD_ENT * EMB,), jnp.float32),
                  jax.ShapeDtypeStruct((PAD_ENT * EMB,), jnp.float32)],
        mesh=mesh,
        scratch_types=[
            pltpu.VMEM((16, GC), jnp.float32),
            pltpu.VMEM((16, GC), jnp.float32),
            pltpu.VMEM((TAILC * EMB,), jnp.float32),
            pltpu.VMEM((GC * EMB,), jnp.float32),
            pltpu.VMEM((16, TAILC), jnp.float32),
            pltpu.SemaphoreType.DMA,
            pltpu.SemaphoreType.DMA,
        ],
        compiler_params=pltpu.CompilerParams(use_tc_tiling_on_sc=True,
                                             needs_layout_passes=False),
    )
    flat_e, flat_r = relay(ent_emb.T, rel_emb.T)
    lin_e = flat_e.reshape(PAD_ENT, EMB)
    lin_r = flat_r.reshape(PAD_ENT, EMB)
    sc = pl.kernel(
        _sc_body,
        out_type=jax.ShapeDtypeStruct((2, TERMS), jnp.float32),
        mesh=mesh,
        scratch_types=[
            pltpu.VMEM((6, NCH, CHUNK), jnp.int32),
            pltpu.VMEM((TPW, EMB), jnp.float32),
            pltpu.VMEM((TPW, EMB), jnp.float32),
            pltpu.VMEM((TPW, EMB), jnp.float32),
            pltpu.VMEM((TPW, EMB), jnp.float32),
            pltpu.VMEM((TPW, EMB), jnp.float32),
            pltpu.VMEM((TPW, EMB), jnp.float32),
            pltpu.VMEM((TPW,), jnp.float32),
            pltpu.VMEM((TPW,), jnp.float32),
            pltpu.SemaphoreType.DMA,
        ],
        compiler_params=pltpu.CompilerParams(use_tc_tiling_on_sc=False,
                                             needs_layout_passes=False),
    )
    sq = sc(idx, lin_e, lin_r)

    loss = pl.pallas_call(
        _loss_body,
        out_shape=jax.ShapeDtypeStruct((1, 1), jnp.float32),
    )(sq)
    return loss[0, 0]
